# baseline (speedup 1.0000x reference)
import functools

import jax
import jax.numpy as jnp
from jax import lax
from jax.experimental import pallas as pl
from jax.experimental.pallas import tpu as pltpu
from jax.experimental.pallas import tpu_sc as plsc

NC = 2
NS = 16
L = 16
NW = NC * NS

BATCH = 16384
D = 32
N_ROWS = 1000000
PACK = 4
N_PACKED = N_ROWS // PACK
B_PER_W = BATCH // NW
CHUNK = 128
NCHUNK = B_PER_W // CHUNK
NBUF = 2
GPC = CHUNK // L

TB = 16384
SUB = TB // PACK
SW = 14
SS = 12
REPACK_GRID = (N_ROWS + TB - 1) // TB
N_PACKED_PAD = REPACK_GRID * SUB


def _repack_body(x_ref, o_ref):
  x = x_ref[...]
  eye = jnp.eye(D, dtype=jnp.float32)
  t = lax.dot_general(x, eye, (((0,), (0,)), ((), ())),
                      preferred_element_type=jnp.float32)
  o_ref[...] = jnp.concatenate(
      [t[q * SUB:(q + 1) * SUB, :] for q in range(PACK)], axis=1)


def _repack(table_t):
  return pl.pallas_call(
      _repack_body,
      grid=(REPACK_GRID,),
      in_specs=[pl.BlockSpec((D, TB), lambda i: (0, i))],
      out_specs=pl.BlockSpec((SUB, PACK * D), lambda i: (i, 0)),
      out_shape=jax.ShapeDtypeStruct((N_PACKED_PAD, PACK * D), jnp.float32),
  )(table_t)


def _sc_body(samp_hbm, user_hbm, book_hbm, out_hbm,
             samp_v, idx_u, idx_b, col_u, col_b, u_rows, b_rows, out_v,
             sem0, sem1):
  wid = lax.axis_index("s") * NC + lax.axis_index("c")
  base = wid * B_PER_W

  pltpu.sync_copy(samp_hbm.at[pl.ds(base * 2, 2 * B_PER_W)], samp_v)

  iota = lax.iota(jnp.int32, L)
  for j in range(NCHUNK):
    for gg in range(GPC):
      g = j * GPC + gg
      pos = 2 * L * g + 2 * iota
      uidx = plsc.load_gather(samp_v, [pos])
      bidx = plsc.load_gather(samp_v, [pos + 1])
      sl = pl.ds(gg * L, L)
      idx_u[j, sl] = (lax.shift_left(lax.shift_right_logical(uidx, SW), SS)
                      + jnp.bitwise_and(uidx, SUB - 1))
      idx_b[j, sl] = (lax.shift_left(lax.shift_right_logical(bidx, SW), SS)
                      + jnp.bitwise_and(bidx, SUB - 1))
      col_u[j, sl] = lax.shift_left(
          jnp.bitwise_and(lax.shift_right_logical(uidx, SS), 3), 5)
      col_b[j, sl] = lax.shift_left(
          jnp.bitwise_and(lax.shift_right_logical(bidx, SS), 3), 5)

  sems = [sem0, sem1]
  descs = {}

  def fire(j):
    s = sems[j % NBUF]
    descs[j] = (
        pltpu.async_copy(user_hbm.at[idx_u.at[j]], u_rows.at[j % NBUF], s),
        pltpu.async_copy(book_hbm.at[idx_b.at[j]], b_rows.at[j % NBUF], s),
    )

  fire(0)
  fire(1)

  for j in range(NCHUNK):
    du, db = descs[j]
    du.wait()
    db.wait()
    uc = u_rows.at[j % NBUF]
    bc = b_rows.at[j % NBUF]
    for gg in range(GPC):
      row = gg * L + iota
      cu = col_u[j, pl.ds(gg * L, L)]
      cb = col_b[j, pl.ds(gg * L, L)]
      acc = jnp.zeros((L,), jnp.float32)
      for d in range(D):
        acc = acc + (plsc.load_gather(uc, [row, cu + d]) *
                     plsc.load_gather(bc, [row, cb + d]))
      res = 5.5 / (1.0 + jnp.exp(-acc))
      out_v[pl.ds((j * GPC + gg) * L, L)] = res
    if j + NBUF < NCHUNK:
      fire(j + NBUF)

  pltpu.sync_copy(out_v, out_hbm.at[pl.ds(base, B_PER_W)])


def _make_sc_kernel():
  mesh = plsc.VectorSubcoreMesh(
      core_axis_name="c", subcore_axis_name="s",
      num_cores=NC, num_subcores=NS)
  return pl.kernel(
      _sc_body,
      out_type=jax.ShapeDtypeStruct((BATCH,), jnp.float32),
      mesh=mesh,
      scratch_types=[
          pltpu.VMEM((2 * B_PER_W,), jnp.int32),
          pltpu.VMEM((NCHUNK, CHUNK), jnp.int32),
          pltpu.VMEM((NCHUNK, CHUNK), jnp.int32),
          pltpu.VMEM((NCHUNK, CHUNK), jnp.int32),
          pltpu.VMEM((NCHUNK, CHUNK), jnp.int32),
          pltpu.VMEM((NBUF, CHUNK, PACK * D), jnp.float32),
          pltpu.VMEM((NBUF, CHUNK, PACK * D), jnp.float32),
          pltpu.VMEM((B_PER_W,), jnp.float32),
          pltpu.SemaphoreType.DMA,
          pltpu.SemaphoreType.DMA,
      ],
      compiler_params=pltpu.CompilerParams(needs_layout_passes=False),
  )


@jax.jit
def kernel(samples, user_embedding, book_embedding):
  samp_flat = samples.reshape(-1).astype(jnp.int32)
  u2 = _repack(user_embedding.T)
  b2 = _repack(book_embedding.T)
  return _make_sc_kernel()(samp_flat, u2, b2)


# speedup vs baseline: 1.0098x; 1.0098x over previous
import functools

import jax
import jax.numpy as jnp
from jax import lax
from jax.experimental import pallas as pl
from jax.experimental.pallas import tpu as pltpu
from jax.experimental.pallas import tpu_sc as plsc

NC = 2
NS = 16
L = 16
NW = NC * NS

BATCH = 16384
D = 32
N_ROWS = 1000000
PACK = 4
N_PACKED = N_ROWS // PACK
B_PER_W = BATCH // NW
CHUNK = 128
NCHUNK = B_PER_W // CHUNK
NBUF = 2
GPC = CHUNK // L

TB = 32768
SUB = TB // PACK
SW = 15
SS = 13
REPACK_GRID = (N_ROWS + TB - 1) // TB
N_PACKED_PAD = REPACK_GRID * SUB


def _repack_body(x_ref, o_ref):
  x = x_ref[...]
  eye = jnp.eye(D, dtype=jnp.float32)
  t = lax.dot_general(x, eye, (((0,), (0,)), ((), ())),
                      preferred_element_type=jnp.float32)
  o_ref[...] = jnp.concatenate(
      [t[q * SUB:(q + 1) * SUB, :] for q in range(PACK)], axis=1)


def _repack(table_t):
  return pl.pallas_call(
      _repack_body,
      grid=(REPACK_GRID,),
      in_specs=[pl.BlockSpec((D, TB), lambda i: (0, i))],
      out_specs=pl.BlockSpec((SUB, PACK * D), lambda i: (i, 0)),
      out_shape=jax.ShapeDtypeStruct((N_PACKED_PAD, PACK * D), jnp.float32),
  )(table_t)


def _sc_body(samp_hbm, user_hbm, book_hbm, out_hbm,
             samp_v, idx_u, idx_b, col_u, col_b, u_rows, b_rows, out_v,
             sem0, sem1):
  wid = lax.axis_index("s") * NC + lax.axis_index("c")
  base = wid * B_PER_W

  pltpu.sync_copy(samp_hbm.at[pl.ds(base * 2, 2 * B_PER_W)], samp_v)

  iota = lax.iota(jnp.int32, L)
  for j in range(NCHUNK):
    for gg in range(GPC):
      g = j * GPC + gg
      pos = 2 * L * g + 2 * iota
      uidx = plsc.load_gather(samp_v, [pos])
      bidx = plsc.load_gather(samp_v, [pos + 1])
      sl = pl.ds(gg * L, L)
      idx_u[j, sl] = (lax.shift_left(lax.shift_right_logical(uidx, SW), SS)
                      + jnp.bitwise_and(uidx, SUB - 1))
      idx_b[j, sl] = (lax.shift_left(lax.shift_right_logical(bidx, SW), SS)
                      + jnp.bitwise_and(bidx, SUB - 1))
      col_u[j, sl] = lax.shift_left(
          jnp.bitwise_and(lax.shift_right_logical(uidx, SS), 3), 5)
      col_b[j, sl] = lax.shift_left(
          jnp.bitwise_and(lax.shift_right_logical(bidx, SS), 3), 5)

  sems = [sem0, sem1]
  descs = {}

  def fire(j):
    s = sems[j % NBUF]
    descs[j] = (
        pltpu.async_copy(user_hbm.at[idx_u.at[j]], u_rows.at[j % NBUF], s),
        pltpu.async_copy(book_hbm.at[idx_b.at[j]], b_rows.at[j % NBUF], s),
    )

  fire(0)
  fire(1)

  for j in range(NCHUNK):
    du, db = descs[j]
    du.wait()
    db.wait()
    uc = u_rows.at[j % NBUF]
    bc = b_rows.at[j % NBUF]
    for gg in range(GPC):
      row = gg * L + iota
      cu = col_u[j, pl.ds(gg * L, L)]
      cb = col_b[j, pl.ds(gg * L, L)]
      acc = jnp.zeros((L,), jnp.float32)
      for d in range(D):
        acc = acc + (plsc.load_gather(uc, [row, cu + d]) *
                     plsc.load_gather(bc, [row, cb + d]))
      res = 5.5 / (1.0 + jnp.exp(-acc))
      out_v[pl.ds((j * GPC + gg) * L, L)] = res
    if j + NBUF < NCHUNK:
      fire(j + NBUF)

  pltpu.sync_copy(out_v, out_hbm.at[pl.ds(base, B_PER_W)])


def _make_sc_kernel():
  mesh = plsc.VectorSubcoreMesh(
      core_axis_name="c", subcore_axis_name="s",
      num_cores=NC, num_subcores=NS)
  return pl.kernel(
      _sc_body,
      out_type=jax.ShapeDtypeStruct((BATCH,), jnp.float32),
      mesh=mesh,
      scratch_types=[
          pltpu.VMEM((2 * B_PER_W,), jnp.int32),
          pltpu.VMEM((NCHUNK, CHUNK), jnp.int32),
          pltpu.VMEM((NCHUNK, CHUNK), jnp.int32),
          pltpu.VMEM((NCHUNK, CHUNK), jnp.int32),
          pltpu.VMEM((NCHUNK, CHUNK), jnp.int32),
          pltpu.VMEM((NBUF, CHUNK, PACK * D), jnp.float32),
          pltpu.VMEM((NBUF, CHUNK, PACK * D), jnp.float32),
          pltpu.VMEM((B_PER_W,), jnp.float32),
          pltpu.SemaphoreType.DMA,
          pltpu.SemaphoreType.DMA,
      ],
      compiler_params=pltpu.CompilerParams(needs_layout_passes=False),
  )


@jax.jit
def kernel(samples, user_embedding, book_embedding):
  samp_flat = samples.reshape(-1).astype(jnp.int32)
  u2 = _repack(user_embedding.T)
  b2 = _repack(book_embedding.T)
  return _make_sc_kernel()(samp_flat, u2, b2)
